# baseline (speedup 1.0000x reference)
import functools
import jax
import jax.numpy as jnp
from jax import lax
from jax.experimental import pallas as pl
from jax.experimental.pallas import tpu as pltpu
from jax.experimental.pallas import tpu_sc as plsc

_F = 4
_L = 8
_DELTA_AGG = 0.5
_DELTA_DIS = 1.5
_RC = 8
_CW = 128

_NC, _NS = 2, 16
_NW = _NC * _NS
_CH = 8192


def _fold(x):
    return (x[:, 0:128] + x[:, 128:256]) + (x[:, 256:384] + x[:, 384:512])


def _make_sc_counts(total):
    span = total // _NW
    nch = span // _CH
    mesh = plsc.VectorSubcoreMesh(core_axis_name="c", subcore_axis_name="s")

    @functools.partial(
        pl.kernel,
        mesh=mesh,
        out_type=jax.ShapeDtypeStruct((_NW, 8, 16), jnp.float32),
        scratch_types=[
            pltpu.VMEM((_CH,), jnp.int32),
            pltpu.VMEM((_CH,), jnp.float32),
            pltpu.VMEM((8, 16), jnp.float32),
        ],
    )
    def sc_counts(inst_hbm, tm_hbm, out_hbm, inst_v, tm_v, out_v):
        wid = lax.axis_index("s") * _NC + lax.axis_index("c")
        base = wid * span
        acc = tuple(jnp.zeros((16,), jnp.float32) for _ in range(7))
        for ch in range(nch):
            pltpu.sync_copy(inst_hbm.at[pl.ds(base + ch * _CH, _CH)], inst_v)
            pltpu.sync_copy(tm_hbm.at[pl.ds(base + ch * _CH, _CH)], tm_v)

            def step(k, carry):
                iv = inst_v[pl.ds(k * 16, 16)]
                tv = tm_v[pl.ds(k * 16, 16)]
                ifl = jnp.where(tv > 0.5, iv, 0)
                return tuple(
                    a + jnp.where(ifl == (c + 1), 1.0, 0.0)
                    for c, a in enumerate(carry)
                )

            acc = lax.fori_loop(0, _CH // 16, step, acc)
        out_v[0, :] = jnp.zeros((16,), jnp.float32)
        for c in range(1, 8):
            out_v[c, :] = acc[c - 1]
        pltpu.sync_copy(out_v, out_hbm.at[wid])

    return sc_counts


def _tc_body(inst_ref, kern_ref, tm_ref, emb_ref, out_ref):
    nchunks = inst_ref.shape[1]
    zero = jnp.zeros((_RC, _CW), jnp.float32)

    def load_ik(i):
        inst = inst_ref[0, i]
        kn = kern_ref[0, i] > 0.5
        tm = tm_ref[0, i] > 0.5
        return jnp.where(tm & kn, inst, 0)

    acc = {}
    for c in range(1, 5):
        acc[c] = [zero] * (_F + 1)
    for i in range(nchunks):
        ik = load_ik(i)
        e = [emb_ref[0, f, i] for f in range(_F)]
        for c in range(1, 5):
            mk = (ik == c).astype(jnp.float32)
            acc[c][_F] = acc[c][_F] + _fold(mk)
            for f in range(_F):
                acc[c][f] = acc[c][f] + _fold(e[f] * mk)
    for c in range(5, 8):
        acc[c] = [zero] * (_F + 1)
    for i in range(nchunks):
        ik = load_ik(i)
        e = [emb_ref[0, f, i] for f in range(_F)]
        for c in range(5, 8):
            mk = (ik == c).astype(jnp.float32)
            acc[c][_F] = acc[c][_F] + _fold(mk)
            for f in range(_F):
                acc[c][f] = acc[c][f] + _fold(e[f] * mk)

    npix = jnp.float32(inst_ref.shape[1] * inst_ref.shape[2] * inst_ref.shape[3])
    cnt_k = [jnp.float32(0.0)]
    sums = [[jnp.float32(0.0)] * _F]
    for c in range(1, _L):
        cnt_k.append(jnp.sum(acc[c][_F]))
        sums.append([jnp.sum(acc[c][f]) for f in range(_F)])
    cnt_k[0] = npix
    for c in range(1, _L):
        cnt_k[0] = cnt_k[0] - cnt_k[c]

    present = [(cnt_k[c] > 0.0).astype(jnp.float32) for c in range(_L)]
    num_inst = present[0]
    for c in range(1, _L):
        num_inst = num_inst + present[c]
    run = jnp.float32(0.0)
    valid = []
    for c in range(_L):
        run = run + present[c]
        valid.append(present[c] * (run - 1.0 >= 1.0).astype(jnp.float32))

    m = [[jnp.float32(0.0)] * _F]
    for c in range(1, _L):
        denom = jnp.maximum(cnt_k[c], 1.0)
        m.append([sums[c][f] / denom for f in range(_F)])

    agg = {c: zero for c in range(1, _L)}
    for i in range(nchunks):
        inst = inst_ref[0, i]
        tm = tm_ref[0, i] > 0.5
        ifl = jnp.where(tm, inst, 0)
        e = [emb_ref[0, f, i] for f in range(_F)]
        cm = [None] + [ifl == c for c in range(1, _L)]
        msel = [jnp.where(cm[1], m[1][f], 0.0) for f in range(_F)]
        for c in range(2, _L):
            msel = [jnp.where(cm[c], m[c][f], msel[f]) for f in range(_F)]
        dd0 = e[0] - msel[0]
        d2 = dd0 * dd0
        for f in range(1, _F):
            ddf = e[f] - msel[f]
            d2 = d2 + ddf * ddf
        dist = jnp.sqrt(d2)
        t = jnp.maximum(dist - _DELTA_AGG, 0.0)
        v = jnp.log(t * t + 1.0)
        for c in range(1, _L):
            agg[c] = agg[c] + _fold(jnp.where(cm[c], v, 0.0))

    dsq_list = []
    pm_list = []
    for i in range(_L):
        for j in range(i + 1, _L):
            dsq = jnp.float32(0.0)
            for f in range(_F):
                dd = m[i][f] - m[j][f]
                dsq = dsq + dd * dd
            dsq_list.append(dsq)
            pm_list.append(valid[i] * valid[j])
    msq_list = []
    for c in range(_L):
        msq = jnp.float32(0.0)
        for f in range(_F):
            msq = msq + m[c][f] * m[c][f]
        msq_list.append(msq)
    npair = len(dsq_list)
    sq = jnp.stack(dsq_list + msq_list)
    d = jnp.where(sq == 0.0, 0.0, jnp.sqrt(jnp.where(sq == 0.0, 1.0, sq)))
    lane36 = jax.lax.iota(jnp.int32, npair + _L)
    is_pair = lane36 < npair
    tdis = jnp.maximum(2.0 * _DELTA_DIS - d, 0.0)
    v36 = jnp.log(jnp.where(is_pair, tdis * tdis + 1.0, d + 1.0))
    pmv = jnp.stack(pm_list + [jnp.float32(0.0)] * _L)
    dis_num = 2.0 * jnp.sum(v36 * pmv)
    pm_sum = 2.0 * jnp.sum(pmv)
    reg_sum = jnp.sum(jnp.where(is_pair, 0.0, v36))
    l_dis = dis_num / jnp.maximum(pm_sum, 1.0)
    l_reg = reg_sum / jnp.maximum(num_inst, 1.0) * 0.001
    disreg = l_dis + l_reg

    li = jax.lax.broadcasted_iota(jnp.int32, (1, 16), 1)
    vec = jnp.zeros((1, 16), jnp.float32)
    for c in range(1, _L):
        vec = jnp.where(li == c, valid[c] * jnp.sum(agg[c]), vec)
    vec = jnp.where(li == 8, num_inst, vec)
    vec = jnp.where(li == 9, disreg, vec)
    out_ref[0] = vec


def _combine_body(tcv_ref, fc_ref, out_ref):
    tcv = tcv_ref[...]
    fc = fc_ref[...]
    ratio = tcv / jnp.maximum(fc, 1.0)
    li = jax.lax.broadcasted_iota(jnp.int32, (8, 16), 1)
    lagg_sum = jnp.sum(jnp.where(li < 8, ratio, 0.0), axis=1)
    ni = jnp.sum(jnp.where(li == 8, tcv, 0.0), axis=1)
    disreg = jnp.sum(jnp.where(li == 9, tcv, 0.0), axis=1)
    l_agg = lagg_sum / jnp.maximum(ni - 1.0, 1.0)
    total = jnp.where(ni <= 1.0, 0.0, l_agg + disreg)
    out_ref[...] = total[:, None] + jnp.zeros((8, 128), jnp.float32)


def kernel(emb, instance, kernel, training_mask):
    B, F, H, W = emb.shape
    nch = H // _RC
    instance = instance.astype(jnp.int32)
    inst_r = instance.reshape(B, nch, _RC, W)
    kern_r = kernel.reshape(B, nch, _RC, W)
    tm_r = training_mask.reshape(B, nch, _RC, W)
    emb_r = emb.reshape(B, F, nch, _RC, W)

    sc_counts = _make_sc_counts(B * H * W)
    scp = sc_counts(instance.reshape(-1), training_mask.reshape(-1))

    tcv = pl.pallas_call(
        _tc_body,
        grid=(B,),
        in_specs=[
            pl.BlockSpec((1, nch, _RC, W), lambda b: (b, 0, 0, 0)),
            pl.BlockSpec((1, nch, _RC, W), lambda b: (b, 0, 0, 0)),
            pl.BlockSpec((1, nch, _RC, W), lambda b: (b, 0, 0, 0)),
            pl.BlockSpec((1, F, nch, _RC, W), lambda b: (b, 0, 0, 0, 0)),
        ],
        out_specs=pl.BlockSpec((1, 1, 16), lambda b: (b, 0, 0)),
        out_shape=jax.ShapeDtypeStruct((B, 1, 16), jnp.float32),
        compiler_params=pltpu.CompilerParams(
            dimension_semantics=("arbitrary",),
        ),
    )(inst_r, kern_r, tm_r, emb_r)

    fc8 = scp.reshape(B, _NW // B, 8, 16).sum(axis=(1, 3))
    fc = jnp.pad(fc8, ((0, 0), (0, 8)))
    out = pl.pallas_call(
        _combine_body,
        out_shape=jax.ShapeDtypeStruct((B, 128), jnp.float32),
    )(tcv.reshape(B, 16), fc)
    return out[:, 0]


# speedup vs baseline: 1.3634x; 1.3634x over previous
import jax
import jax.numpy as jnp
from jax.experimental import pallas as pl
from jax.experimental.pallas import tpu as pltpu

_F = 4
_L = 8
_DELTA_AGG = 0.5
_DELTA_DIS = 1.5
_W_AGG = 1.0
_W_DIS = 1.0
_RC = 8
_CW = 128


def _scalar_safe_sqrt(sq):
    safe = jnp.where(sq == 0.0, 1.0, sq)
    return jnp.where(sq == 0.0, 0.0, jnp.sqrt(safe))


def _fold(x):
    return (x[:, 0:128] + x[:, 128:256]) + (x[:, 256:384] + x[:, 384:512])


def _loss_body(inst_ref, kern_ref, tm_ref, emb_ref, out_ref):
    nchunks = inst_ref.shape[1]
    zero = jnp.zeros((_RC, _CW), jnp.float32)

    def load_ik(i):
        inst = inst_ref[0, i]
        kn = kern_ref[0, i] > 0.5
        tm = tm_ref[0, i] > 0.5
        return jnp.where(tm & kn, inst, 0)

    acc = {}
    for c in range(1, _L):
        acc[c] = [zero] * (_F + 1)
    for i in range(nchunks):
        ik = load_ik(i)
        e = [emb_ref[0, f, i] for f in range(_F)]
        for c in range(1, _L):
            mk = (ik == c).astype(jnp.float32)
            acc[c][_F] = acc[c][_F] + _fold(mk)
            for f in range(_F):
                acc[c][f] = acc[c][f] + _fold(e[f] * mk)

    npix = jnp.float32(inst_ref.shape[1] * inst_ref.shape[2] * inst_ref.shape[3])
    cnt_k = [jnp.float32(0.0)]
    sums = [[jnp.float32(0.0)] * _F]
    for c in range(1, _L):
        cnt_k.append(jnp.sum(acc[c][_F]))
        sums.append([jnp.sum(acc[c][f]) for f in range(_F)])
    cnt_k[0] = npix
    for c in range(1, _L):
        cnt_k[0] = cnt_k[0] - cnt_k[c]

    present = [(cnt_k[c] > 0.0).astype(jnp.float32) for c in range(_L)]
    num_inst = present[0]
    for c in range(1, _L):
        num_inst = num_inst + present[c]
    run = jnp.float32(0.0)
    valid = []
    for c in range(_L):
        run = run + present[c]
        rank = run - 1.0
        valid.append(present[c] * (rank >= 1.0).astype(jnp.float32))

    m = [[jnp.float32(0.0)] * _F]
    for c in range(1, _L):
        denom = jnp.maximum(cnt_k[c], 1.0)
        m.append([sums[c][f] / denom for f in range(_F)])

    agg = {c: zero for c in range(1, _L)}
    fcnt = {c: zero for c in range(1, _L)}
    for i in range(nchunks):
        inst = inst_ref[0, i]
        tm = tm_ref[0, i] > 0.5
        ifl = jnp.where(tm, inst, 0)
        e = [emb_ref[0, f, i] for f in range(_F)]
        cm = [None] + [ifl == c for c in range(1, _L)]
        msel = [jnp.where(cm[1], m[1][f], 0.0) for f in range(_F)]
        for c in range(2, _L):
            msel = [jnp.where(cm[c], m[c][f], msel[f]) for f in range(_F)]
        dd0 = e[0] - msel[0]
        d2 = dd0 * dd0
        for f in range(1, _F):
            ddf = e[f] - msel[f]
            d2 = d2 + ddf * ddf
        dist = jnp.sqrt(d2)
        t = jnp.maximum(dist - _DELTA_AGG, 0.0)
        v = jnp.log(t * t + 1.0)
        for c in range(1, _L):
            agg[c] = agg[c] + _fold(jnp.where(cm[c], v, 0.0))
            fcnt[c] = fcnt[c] + _fold(cm[c].astype(jnp.float32))
    l_agg = jnp.float32(0.0)
    for c in range(1, _L):
        l_agg = l_agg + valid[c] * jnp.sum(agg[c]) / jnp.maximum(
            jnp.sum(fcnt[c]), 1.0)
    l_agg = l_agg / jnp.maximum(num_inst - 1.0, 1.0)

    dsq_list = []
    pm_list = []
    for i in range(_L):
        for j in range(i + 1, _L):
            dsq = jnp.float32(0.0)
            for f in range(_F):
                dd = m[i][f] - m[j][f]
                dsq = dsq + dd * dd
            dsq_list.append(dsq)
            pm_list.append(valid[i] * valid[j])
    msq_list = []
    for c in range(_L):
        msq = jnp.float32(0.0)
        for f in range(_F):
            msq = msq + m[c][f] * m[c][f]
        msq_list.append(msq)

    npair = len(dsq_list)
    sq = jnp.stack(dsq_list + msq_list)
    d = jnp.where(sq == 0.0, 0.0,
                  jnp.sqrt(jnp.where(sq == 0.0, 1.0, sq)))
    lane = jax.lax.iota(jnp.int32, npair + _L)
    is_pair = lane < npair
    tdis = jnp.maximum(2.0 * _DELTA_DIS - d, 0.0)
    arg = jnp.where(is_pair, tdis * tdis + 1.0, d + 1.0)
    v = jnp.log(arg)
    pmv = jnp.stack(pm_list + [jnp.float32(0.0)] * _L)
    dis_num = 2.0 * jnp.sum(v * pmv)
    pm_sum = 2.0 * jnp.sum(pmv)
    reg_sum = jnp.sum(jnp.where(is_pair, 0.0, v))
    l_dis = dis_num / jnp.maximum(pm_sum, 1.0)
    l_reg = reg_sum / jnp.maximum(num_inst, 1.0) * 0.001

    total = _W_AGG * l_agg + _W_DIS * l_dis + l_reg
    out_ref[0, 0, 0] = jnp.where(num_inst <= 1.0, 0.0, total)


def kernel(emb, instance, kernel, training_mask):
    B, F, H, W = emb.shape
    nch = H // _RC
    instance = instance.astype(jnp.int32).reshape(B, nch, _RC, W)
    kernel = kernel.reshape(B, nch, _RC, W)
    training_mask = training_mask.reshape(B, nch, _RC, W)
    emb = emb.reshape(B, F, nch, _RC, W)
    out = pl.pallas_call(
        _loss_body,
        grid=(B,),
        in_specs=[
            pl.BlockSpec((1, nch, _RC, W), lambda b: (b, 0, 0, 0)),
            pl.BlockSpec((1, nch, _RC, W), lambda b: (b, 0, 0, 0)),
            pl.BlockSpec((1, nch, _RC, W), lambda b: (b, 0, 0, 0)),
            pl.BlockSpec((1, F, nch, _RC, W), lambda b: (b, 0, 0, 0, 0)),
        ],
        out_specs=pl.BlockSpec(
            (1, 1, 1), lambda b: (b, 0, 0), memory_space=pltpu.SMEM
        ),
        out_shape=jax.ShapeDtypeStruct((B, 1, 1), jnp.float32),
        compiler_params=pltpu.CompilerParams(
            dimension_semantics=("arbitrary",),
        ),
    )(instance, kernel, training_mask, emb)
    return out.reshape(B)
